# baseline (speedup 1.0000x reference)
import jax
import jax.numpy as jnp
from jax.experimental import pallas as pl

_NUM_EDGES = 320000
_NUM_PIV = 100
_STRIDE = 100


def _body(x_ref, o_ref):
    val = x_ref[:, 0, :, :]
    onehot = (jax.lax.broadcasted_iota(jnp.int32, (1, 1, _STRIDE), 2) == 0)
    o_ref[:, :] = jnp.sum(jnp.where(onehot, val, 0.0), axis=-1)


def kernel(x):
    x4 = x.reshape(128, 33, _STRIDE, _STRIDE)
    return pl.pallas_call(
        _body,
        grid=(1,),
        in_specs=[pl.BlockSpec((128, 1, _STRIDE, _STRIDE),
                               lambda i: (0, _NUM_EDGES // (_STRIDE * _STRIDE), 0, 0))],
        out_specs=pl.BlockSpec((128, _NUM_PIV), lambda i: (0, 0)),
        out_shape=jax.ShapeDtypeStruct((128, _NUM_PIV), jnp.float32),
    )(x4)
